# baseline (speedup 1.0000x reference)
import functools

import jax
import jax.numpy as jnp
from jax import lax
from jax.experimental import pallas as pl
from jax.experimental.pallas import tpu as pltpu
from jax.experimental.pallas import tpu_sc as plsc

NC, NS, L = 2, 16, 16
NW = NC * NS

N_NODES = 100000
D = 128
N0, N1, N2 = 2048, 32768, 524288
E0, E1 = 32768, 524288
NCLS = 64

EB = 128
EBA = 128
NBUF = 4
SKEW = 3
CW = 16
PW = (48, 48, 32)
TR = 256

_MESH = plsc.VectorSubcoreMesh(
    core_axis_name="c", subcore_axis_name="s", num_cores=NC, num_subcores=NS)
_SC_PARAMS = pltpu.CompilerParams(
    use_tc_tiling_on_sc=False, needs_layout_passes=False)


def _zero_rows(buf, n, w):
    zv = jnp.zeros((L,), jnp.float32)

    def zr(i, carry):
        for j in range(w // L):
            buf[i, pl.ds(j * L, L)] = zv
        return carry

    lax.fori_loop(0, n, zr, 0)


def _fill_ones(buf, n, w):
    ov = jnp.ones((L,), jnp.float32)

    def fr(i, carry):
        for j in range(w // L):
            buf[i, pl.ds(j * L, L)] = ov
        return carry

    lax.fori_loop(0, n, fr, 0)


def _make_pass_a(W, EB, compose, with_cnt):
    out_type = [jax.ShapeDtypeStruct((NC, N1, W), jnp.float32)]
    if compose:
        out_type.append(jax.ShapeDtypeStruct((E1,), jnp.int32))
    if with_cnt:
        out_type.append(jax.ShapeDtypeStruct((NC, N1, CW), jnp.float32))

    scratch = [pltpu.VMEM_SHARED((N1, W), jnp.float32)]
    if with_cnt:
        scratch.append(pltpu.VMEM_SHARED((N1, CW), jnp.float32))
    scratch += [pltpu.VMEM((EB,), jnp.int32)] * NBUF
    scratch += [pltpu.VMEM((EB,), jnp.int32)] * NBUF
    scratch += [pltpu.VMEM((EB,), jnp.int32)] * NBUF
    scratch += [pltpu.VMEM((EB, W), jnp.float32)] * NBUF
    if with_cnt:
        scratch.append(pltpu.VMEM((EB, CW), jnp.float32))
    scratch += [pltpu.SemaphoreType.DMA] * (3 * NBUF)
    if with_cnt:
        scratch += [pltpu.SemaphoreType.DMA] * NBUF

    def body(*refs):
        it = iter(refs)
        ftab = next(it)
        if compose:
            srcr = next(it)
        else:
            idx1r = next(it)
        dstr = next(it)
        if compose:
            samp = next(it)
        s_out = next(it)
        idx1_out = next(it) if compose else None
        cnt_out = next(it) if with_cnt else None
        acc = next(it)
        cacc = next(it) if with_cnt else None
        srcb = [next(it) for _ in range(NBUF)]
        idxb = [next(it) for _ in range(NBUF)]
        dstb = [next(it) for _ in range(NBUF)]
        rows = [next(it) for _ in range(NBUF)]
        onesb = next(it) if with_cnt else None
        isem = [next(it) for _ in range(NBUF)]
        gsem = [next(it) for _ in range(NBUF)]
        ssem = [next(it) for _ in range(NBUF)]
        csem = [next(it) for _ in range(NBUF)] if with_cnt else None

        c = lax.axis_index("c")
        s = lax.axis_index("s")
        wid = s * NC + c
        rpt = N1 // NS

        _zero_rows(rows[0], EB, W)

        def zacc(k, carry):
            pltpu.sync_copy(rows[0], acc.at[pl.ds(s * rpt + k * EB, EB), :])
            return carry

        lax.fori_loop(0, rpt // EB, zacc, 0)
        if with_cnt:
            _zero_rows(onesb, EB, CW)

            def zcnt(k, carry):
                pltpu.sync_copy(onesb, cacc.at[pl.ds(s * rpt + k * EB, EB), :])
                return carry

            lax.fori_loop(0, rpt // EB, zcnt, 0)
            _fill_ones(onesb, EB, CW)
        plsc.subcore_barrier()

        per_w = E1 // NW
        base = wid * per_w
        rs = min(SKEW + 1, NBUF - 1) if compose else SKEW

        def load_b(i, b):
            off = base + i * EB
            if compose:
                pltpu.sync_copy(srcr.at[pl.ds(off, EB)], srcb[b])
                pltpu.async_copy(samp.at[srcb[b]], idxb[b], isem[b])
            else:
                pltpu.sync_copy(idx1r.at[pl.ds(off, EB)], idxb[b])
            pltpu.sync_copy(dstr.at[pl.ds(off, EB)], dstb[b])

        def issue_rows(i, b):
            if compose:
                pltpu.make_async_copy(samp.at[srcb[b]], idxb[b], isem[b]).wait()
                pltpu.sync_copy(idxb[b], idx1_out.at[pl.ds(base + i * EB, EB)])
            pltpu.async_copy(ftab.at[idxb[b]], rows[b], gsem[b])

        def wait_scatter(b):
            pltpu.make_async_copy(rows[b], acc.at[dstb[b]], ssem[b]).wait()
            if with_cnt:
                pltpu.make_async_copy(onesb, cacc.at[dstb[b]], csem[b]).wait()

        def drain_gather_issue_scatter(pb):
            pltpu.make_async_copy(ftab.at[idxb[pb]], rows[pb], gsem[pb]).wait()
            pltpu.async_copy(rows[pb], acc.at[dstb[pb]], ssem[pb], add=True)
            if with_cnt:
                pltpu.async_copy(onesb, cacc.at[dstb[pb]], csem[pb], add=True)

        def grp(g, carry):
            for b in range(NBUF):
                t = g * NBUF + b

                @pl.when(g >= 1)
                def _(b=b):
                    wait_scatter(b)

                load_b(t, b)
                if compose:
                    rb = (b - 1) % NBUF
                    if b >= 1:
                        issue_rows(t - 1, rb)
                    else:
                        @pl.when(g >= 1)
                        def _(t=t, rb=rb):
                            issue_rows(t - 1, rb)
                else:
                    issue_rows(t, b)
                pb = (b - rs) % NBUF
                if b >= rs:
                    drain_gather_issue_scatter(pb)
                else:
                    @pl.when(g >= 1)
                    def _(pb=pb):
                        drain_gather_issue_scatter(pb)
            return carry

        nb = per_w // EB
        lax.fori_loop(0, nb // NBUF, grp, 0)
        if compose:
            issue_rows(nb - 1, (nb - 1) % NBUF)
        for k in range(rs if not compose else rs - 1):
            drain_gather_issue_scatter((nb - (rs - k)) % NBUF)
        if compose:
            drain_gather_issue_scatter((nb - 1) % NBUF)
        for b in range(NBUF):
            wait_scatter(b)
        plsc.subcore_barrier()

        pltpu.sync_copy(acc.at[pl.ds(s * rpt, rpt), :],
                        s_out.at[c, pl.ds(s * rpt, rpt), :])
        if with_cnt:
            pltpu.sync_copy(cacc.at[pl.ds(s * rpt, rpt), :],
                            cnt_out.at[c, pl.ds(s * rpt, rpt), :])

    return pl.kernel(body, out_type=tuple(out_type), mesh=_MESH,
                     scratch_types=tuple(scratch),
                     compiler_params=_SC_PARAMS)


_pass_a1 = _make_pass_a(PW[0], EBA, compose=True, with_cnt=False)
_pass_a2 = _make_pass_a(PW[1], EBA, compose=False, with_cnt=False)
_pass_a3 = _make_pass_a(PW[2], EBA, compose=False, with_cnt=True)


def _kern_b_body(*refs):
    it = iter(refs)
    feat, s1r, s0r, srcr, dstr = (next(it) for _ in range(5))
    s0_out, cnt_out, g1_out, g0_out = (next(it) for _ in range(4))
    acc, cacc, s1b = (next(it) for _ in range(3))
    srcb = [next(it) for _ in range(NBUF)]
    idxb = [next(it) for _ in range(NBUF)]
    dstb = [next(it) for _ in range(NBUF)]
    rows = [next(it) for _ in range(NBUF)]
    rows64, onesb = next(it), next(it)
    gsem = [next(it) for _ in range(NBUF)]
    ssem = [next(it) for _ in range(NBUF)]
    csem = [next(it) for _ in range(NBUF)]

    c = lax.axis_index("c")
    s = lax.axis_index("s")
    wid = s * NC + c
    rpt = N0 // NS

    _zero_rows(rows[0], EB, D)
    pltpu.sync_copy(rows[0], acc.at[pl.ds(s * rpt, rpt), :])
    _zero_rows(onesb, EB, CW)
    pltpu.sync_copy(onesb, cacc.at[pl.ds(s * rpt, rpt), :])
    _fill_ones(onesb, EB, CW)
    pltpu.sync_copy(s1r, s1b)
    plsc.subcore_barrier()

    per_w = E0 // NW
    base = wid * per_w
    nb = per_w // EB

    def load_b(t, b):
        off = base + t * EB
        pltpu.sync_copy(srcr.at[pl.ds(off, EB)], srcb[b])
        for j in range(EB // L):
            iv = plsc.load_gather(s1b, [srcb[b][pl.ds(j * L, L)]])
            idxb[b][pl.ds(j * L, L)] = iv
        pltpu.sync_copy(dstr.at[pl.ds(off, EB)], dstb[b])

    def wait_scatter(b):
        pltpu.make_async_copy(rows[b], acc.at[dstb[b]], ssem[b]).wait()
        pltpu.make_async_copy(onesb, cacc.at[dstb[b]], csem[b]).wait()

    def drain_issue_scatter(pb):
        pltpu.make_async_copy(feat.at[idxb[pb]], rows[pb], gsem[pb]).wait()
        pltpu.async_copy(rows[pb], acc.at[dstb[pb]], ssem[pb], add=True)
        pltpu.async_copy(onesb, cacc.at[dstb[pb]], csem[pb], add=True)

    for g in range(nb // NBUF):
        for b in range(NBUF):
            t = g * NBUF + b
            if g >= 1:
                wait_scatter(b)
            load_b(t, b)
            pltpu.async_copy(feat.at[idxb[b]], rows[b], gsem[b])
            if t >= SKEW:
                drain_issue_scatter((b - SKEW) % NBUF)
    for k in range(SKEW):
        drain_issue_scatter((nb - (SKEW - k)) % NBUF)
    for b in range(NBUF):
        wait_scatter(b)

    g1base = wid * (N1 // NW)

    def g1_issue(t, b):
        pltpu.async_copy(feat.at[s1b.at[pl.ds(g1base + t * EB, EB)]],
                         rows[b], gsem[b])

    def g1_drain(t, pb):
        pltpu.make_async_copy(feat.at[s1b.at[pl.ds(g1base + t * EB, EB)]],
                              rows[pb], gsem[pb]).wait()
        pltpu.sync_copy(rows[pb], g1_out.at[pl.ds(g1base + t * EB, EB), :])

    ng1 = (N1 // NW) // EB
    for t in range(ng1):
        g1_issue(t, t % NBUF)
        if t >= SKEW:
            g1_drain(t - SKEW, (t - SKEW) % NBUF)
    for t in range(ng1 - SKEW, ng1):
        g1_drain(t, t % NBUF)

    g0n = N0 // NW
    r0 = wid * g0n
    pltpu.sync_copy(s0r.at[pl.ds(r0, g0n)], srcb[0].at[pl.ds(0, g0n)])
    pltpu.async_copy(feat.at[srcb[0].at[pl.ds(0, g0n)]], rows64, gsem[0]).wait()
    pltpu.sync_copy(rows64, g0_out.at[pl.ds(r0, g0n), :])

    plsc.subcore_barrier()
    pltpu.sync_copy(acc.at[pl.ds(s * rpt, rpt), :],
                    s0_out.at[c, pl.ds(s * rpt, rpt), :])
    pltpu.sync_copy(cacc.at[pl.ds(s * rpt, rpt), :],
                    cnt_out.at[c, pl.ds(s * rpt, rpt), :])


_kern_b = pl.kernel(
    _kern_b_body,
    out_type=(
        jax.ShapeDtypeStruct((NC, N0, D), jnp.float32),
        jax.ShapeDtypeStruct((NC, N0, CW), jnp.float32),
        jax.ShapeDtypeStruct((N1, D), jnp.float32),
        jax.ShapeDtypeStruct((N0, D), jnp.float32),
    ),
    mesh=_MESH,
    scratch_types=(
        pltpu.VMEM_SHARED((N0, D), jnp.float32),
        pltpu.VMEM_SHARED((N0, CW), jnp.float32),
        pltpu.VMEM((N1,), jnp.int32),
    ) + tuple([pltpu.VMEM((EB,), jnp.int32)] * (3 * NBUF)) + tuple(
        [pltpu.VMEM((EB, D), jnp.float32)] * NBUF) + (
        pltpu.VMEM((N0 // NW, D), jnp.float32),
        pltpu.VMEM((EB, CW), jnp.float32),
    ) + tuple([pltpu.SemaphoreType.DMA] * (3 * NBUF)),
    compiler_params=_SC_PARAMS,
)


def _kern_c_body(*refs):
    it = iter(refs)
    h1r, srcr, dstr, s2_out, acc = (next(it) for _ in range(5))
    srcb = [next(it) for _ in range(NBUF)]
    dstb = [next(it) for _ in range(NBUF)]
    rows = [next(it) for _ in range(NBUF)]
    gsem = [next(it) for _ in range(NBUF)]
    ssem = [next(it) for _ in range(NBUF)]

    c = lax.axis_index("c")
    s = lax.axis_index("s")
    wid = s * NC + c
    rpt = N0 // NS

    _zero_rows(rows[0], EB, D)
    pltpu.sync_copy(rows[0], acc.at[pl.ds(s * rpt, rpt), :])
    plsc.subcore_barrier()

    per_w = E0 // NW
    base = wid * per_w
    nb = per_w // EB

    def wait_scatter(b):
        pltpu.make_async_copy(rows[b], acc.at[dstb[b]], ssem[b]).wait()

    def drain_issue_scatter(pb):
        pltpu.make_async_copy(h1r.at[srcb[pb]], rows[pb], gsem[pb]).wait()
        pltpu.async_copy(rows[pb], acc.at[dstb[pb]], ssem[pb], add=True)

    for g in range(nb // NBUF):
        for b in range(NBUF):
            t = g * NBUF + b
            if g >= 1:
                wait_scatter(b)
            off = base + t * EB
            pltpu.sync_copy(srcr.at[pl.ds(off, EB)], srcb[b])
            pltpu.sync_copy(dstr.at[pl.ds(off, EB)], dstb[b])
            pltpu.async_copy(h1r.at[srcb[b]], rows[b], gsem[b])
            if t >= SKEW:
                drain_issue_scatter((b - SKEW) % NBUF)
    for k in range(SKEW):
        drain_issue_scatter((nb - (SKEW - k)) % NBUF)
    for b in range(NBUF):
        wait_scatter(b)
    plsc.subcore_barrier()
    pltpu.sync_copy(acc.at[pl.ds(s * rpt, rpt), :],
                    s2_out.at[c, pl.ds(s * rpt, rpt), :])


_kern_c = pl.kernel(
    _kern_c_body,
    out_type=(jax.ShapeDtypeStruct((NC, N0, D), jnp.float32),),
    mesh=_MESH,
    scratch_types=(
        pltpu.VMEM_SHARED((N0, D), jnp.float32),
    ) + tuple([pltpu.VMEM((EB,), jnp.int32)] * (2 * NBUF)) + tuple(
        [pltpu.VMEM((EB, D), jnp.float32)] * NBUF) + tuple(
        [pltpu.SemaphoreType.DMA] * (2 * NBUF)),
    compiler_params=_SC_PARAMS,
)


def _tc_h1_body(s1r, s2r, s3r, cr, gr, wtr, br, outr):
    sfull = jnp.concatenate(
        [s1r[0] + s1r[1], s2r[0] + s2r[1], s3r[0] + s3r[1]], axis=-1)
    cnt = jnp.maximum(cr[0, :, 0:1] + cr[1, :, 0:1], 1.0)
    h = jnp.maximum(sfull / cnt + gr[...], 0.0)
    outr[...] = jnp.dot(h, wtr[...], preferred_element_type=jnp.float32) + br[...]


def _tc_h1(s1, s2, s3, cntp, g, wt, bb):
    return pl.pallas_call(
        _tc_h1_body,
        grid=(N1 // TR,),
        in_specs=[
            pl.BlockSpec((NC, TR, PW[0]), lambda i: (0, i, 0)),
            pl.BlockSpec((NC, TR, PW[1]), lambda i: (0, i, 0)),
            pl.BlockSpec((NC, TR, PW[2]), lambda i: (0, i, 0)),
            pl.BlockSpec((NC, TR, CW), lambda i: (0, i, 0)),
            pl.BlockSpec((TR, D), lambda i: (i, 0)),
            pl.BlockSpec((D, D), lambda i: (0, 0)),
            pl.BlockSpec((1, D), lambda i: (0, 0)),
        ],
        out_specs=pl.BlockSpec((TR, D), lambda i: (i, 0)),
        out_shape=jax.ShapeDtypeStruct((N1, D), jnp.float32),
    )(s1, s2, s3, cntp, g, wt, bb)


def _tc_full_body(spr, cr, gr, wtr, br, outr):
    sfull = spr[0] + spr[1]
    cnt = jnp.maximum(cr[0, :, 0:1] + cr[1, :, 0:1], 1.0)
    h = jnp.maximum(sfull / cnt + gr[...], 0.0)
    outr[...] = jnp.dot(h, wtr[...], preferred_element_type=jnp.float32) + br[...]


def _tc_full(sp, cntp, g, wt, bb, dout):
    rows_n = g.shape[0]
    return pl.pallas_call(
        _tc_full_body,
        grid=(rows_n // TR,),
        in_specs=[
            pl.BlockSpec((NC, TR, D), lambda i: (0, i, 0)),
            pl.BlockSpec((NC, TR, CW), lambda i: (0, i, 0)),
            pl.BlockSpec((TR, D), lambda i: (i, 0)),
            pl.BlockSpec((D, dout), lambda i: (0, 0)),
            pl.BlockSpec((1, dout), lambda i: (0, 0)),
        ],
        out_specs=pl.BlockSpec((TR, dout), lambda i: (i, 0)),
        out_shape=jax.ShapeDtypeStruct((rows_n, dout), jnp.float32),
    )(sp, cntp, g, wt, bb)


@jax.jit
def kernel(features, W0, b0, W1, b1, sample0, sample1, sample2,
           block0_src, block0_dst, block1_src, block1_dst):
    f1 = features[:, :PW[0]]
    f2 = features[:, PW[0]:PW[0] + PW[1]]
    f3 = features[:, PW[0] + PW[1]:]
    w0t = W0.T
    w1t = W1.T
    b0r = b0.reshape(1, -1)
    b1r = b1.reshape(1, -1)

    s1p1, idx1 = _pass_a1(f1, block1_src, block1_dst, sample2)
    (s1p2,) = _pass_a2(f2, idx1, block1_dst)
    s1p3, cnt1 = _pass_a3(f3, idx1, block1_dst)

    s0p, cnt0, g1, g0 = _kern_b(features, sample1, sample0,
                                block0_src, block0_dst)

    h1 = _tc_h1(s1p1, s1p2, s1p3, cnt1, g1, w0t, b0r)
    h0 = _tc_full(s0p, cnt0, g0, w0t, b0r, D)

    (s2p,) = _kern_c(h1, block0_src, block0_dst)
    out = _tc_full(s2p, cnt0, h0, w1t, b1r, NCLS)
    return out


# speedup vs baseline: 1.0690x; 1.0690x over previous
import functools

import jax
import jax.numpy as jnp
from jax import lax
from jax.experimental import pallas as pl
from jax.experimental.pallas import tpu as pltpu
from jax.experimental.pallas import tpu_sc as plsc

NC, NS, L = 2, 16, 16
NW = NC * NS

N_NODES = 100000
D = 128
N0, N1, N2 = 2048, 32768, 524288
E0, E1 = 32768, 524288
NCLS = 64

EB = 128
EBA = 128
NBUF = 4
SKEW = 2
CW = 16
PW = (48, 48, 32)
TR = 256

_MESH = plsc.VectorSubcoreMesh(
    core_axis_name="c", subcore_axis_name="s", num_cores=NC, num_subcores=NS)
_SC_PARAMS = pltpu.CompilerParams(
    use_tc_tiling_on_sc=False, needs_layout_passes=False)


def _zero_rows(buf, n, w):
    zv = jnp.zeros((L,), jnp.float32)

    def zr(i, carry):
        for j in range(w // L):
            buf[i, pl.ds(j * L, L)] = zv
        return carry

    lax.fori_loop(0, n, zr, 0)


def _fill_ones(buf, n, w):
    ov = jnp.ones((L,), jnp.float32)

    def fr(i, carry):
        for j in range(w // L):
            buf[i, pl.ds(j * L, L)] = ov
        return carry

    lax.fori_loop(0, n, fr, 0)


def _make_pass_a(W, EB, compose, with_cnt):
    out_type = [jax.ShapeDtypeStruct((NC, N1, W), jnp.float32)]
    if compose:
        out_type.append(jax.ShapeDtypeStruct((E1,), jnp.int32))
    if with_cnt:
        out_type.append(jax.ShapeDtypeStruct((NC, N1, CW), jnp.float32))

    scratch = [pltpu.VMEM_SHARED((N1, W), jnp.float32)]
    if with_cnt:
        scratch.append(pltpu.VMEM_SHARED((N1, CW), jnp.float32))
    scratch += [pltpu.VMEM((EB,), jnp.int32)] * NBUF
    scratch += [pltpu.VMEM((EB,), jnp.int32)] * NBUF
    scratch += [pltpu.VMEM((EB,), jnp.int32)] * NBUF
    scratch += [pltpu.VMEM((EB, W), jnp.float32)] * NBUF
    if with_cnt:
        scratch.append(pltpu.VMEM((EB, CW), jnp.float32))
    scratch += [pltpu.SemaphoreType.DMA] * (3 * NBUF)
    if with_cnt:
        scratch += [pltpu.SemaphoreType.DMA] * NBUF

    def body(*refs):
        it = iter(refs)
        ftab = next(it)
        if compose:
            srcr = next(it)
        else:
            idx1r = next(it)
        dstr = next(it)
        if compose:
            samp = next(it)
        s_out = next(it)
        idx1_out = next(it) if compose else None
        cnt_out = next(it) if with_cnt else None
        acc = next(it)
        cacc = next(it) if with_cnt else None
        srcb = [next(it) for _ in range(NBUF)]
        idxb = [next(it) for _ in range(NBUF)]
        dstb = [next(it) for _ in range(NBUF)]
        rows = [next(it) for _ in range(NBUF)]
        onesb = next(it) if with_cnt else None
        isem = [next(it) for _ in range(NBUF)]
        gsem = [next(it) for _ in range(NBUF)]
        ssem = [next(it) for _ in range(NBUF)]
        csem = [next(it) for _ in range(NBUF)] if with_cnt else None

        c = lax.axis_index("c")
        s = lax.axis_index("s")
        wid = s * NC + c
        rpt = N1 // NS

        _zero_rows(rows[0], EB, W)

        def zacc(k, carry):
            pltpu.sync_copy(rows[0], acc.at[pl.ds(s * rpt + k * EB, EB), :])
            return carry

        lax.fori_loop(0, rpt // EB, zacc, 0)
        if with_cnt:
            _zero_rows(onesb, EB, CW)

            def zcnt(k, carry):
                pltpu.sync_copy(onesb, cacc.at[pl.ds(s * rpt + k * EB, EB), :])
                return carry

            lax.fori_loop(0, rpt // EB, zcnt, 0)
            _fill_ones(onesb, EB, CW)
        plsc.subcore_barrier()

        per_w = E1 // NW
        base = wid * per_w
        rs = min(SKEW + 1, NBUF - 1) if compose else SKEW

        def load_b(i, b):
            off = base + i * EB
            if compose:
                pltpu.sync_copy(srcr.at[pl.ds(off, EB)], srcb[b])
                pltpu.async_copy(samp.at[srcb[b]], idxb[b], isem[b])
            else:
                pltpu.sync_copy(idx1r.at[pl.ds(off, EB)], idxb[b])
            pltpu.sync_copy(dstr.at[pl.ds(off, EB)], dstb[b])

        def issue_rows(i, b):
            if compose:
                pltpu.make_async_copy(samp.at[srcb[b]], idxb[b], isem[b]).wait()
                pltpu.sync_copy(idxb[b], idx1_out.at[pl.ds(base + i * EB, EB)])
            pltpu.async_copy(ftab.at[idxb[b]], rows[b], gsem[b])

        def wait_scatter(b):
            pltpu.make_async_copy(rows[b], acc.at[dstb[b]], ssem[b]).wait()
            if with_cnt:
                pltpu.make_async_copy(onesb, cacc.at[dstb[b]], csem[b]).wait()

        def drain_gather_issue_scatter(pb):
            pltpu.make_async_copy(ftab.at[idxb[pb]], rows[pb], gsem[pb]).wait()
            pltpu.async_copy(rows[pb], acc.at[dstb[pb]], ssem[pb], add=True)
            if with_cnt:
                pltpu.async_copy(onesb, cacc.at[dstb[pb]], csem[pb], add=True)

        def grp(g, carry):
            for b in range(NBUF):
                t = g * NBUF + b

                @pl.when(g >= 1)
                def _(b=b):
                    wait_scatter(b)

                load_b(t, b)
                if compose:
                    rb = (b - 1) % NBUF
                    if b >= 1:
                        issue_rows(t - 1, rb)
                    else:
                        @pl.when(g >= 1)
                        def _(t=t, rb=rb):
                            issue_rows(t - 1, rb)
                else:
                    issue_rows(t, b)
                pb = (b - rs) % NBUF
                if b >= rs:
                    drain_gather_issue_scatter(pb)
                else:
                    @pl.when(g >= 1)
                    def _(pb=pb):
                        drain_gather_issue_scatter(pb)
            return carry

        nb = per_w // EB
        lax.fori_loop(0, nb // NBUF, grp, 0)
        if compose:
            issue_rows(nb - 1, (nb - 1) % NBUF)
        for k in range(rs if not compose else rs - 1):
            drain_gather_issue_scatter((nb - (rs - k)) % NBUF)
        if compose:
            drain_gather_issue_scatter((nb - 1) % NBUF)
        for b in range(NBUF):
            wait_scatter(b)
        plsc.subcore_barrier()

        pltpu.sync_copy(acc.at[pl.ds(s * rpt, rpt), :],
                        s_out.at[c, pl.ds(s * rpt, rpt), :])
        if with_cnt:
            pltpu.sync_copy(cacc.at[pl.ds(s * rpt, rpt), :],
                            cnt_out.at[c, pl.ds(s * rpt, rpt), :])

    return pl.kernel(body, out_type=tuple(out_type), mesh=_MESH,
                     scratch_types=tuple(scratch),
                     compiler_params=_SC_PARAMS)


_pass_a1 = _make_pass_a(PW[0], EBA, compose=True, with_cnt=False)
_pass_a2 = _make_pass_a(PW[1], EBA, compose=False, with_cnt=False)
_pass_a3 = _make_pass_a(PW[2], EBA, compose=False, with_cnt=True)


def _kern_b_body(*refs):
    it = iter(refs)
    feat, s1r, s0r, srcr, dstr = (next(it) for _ in range(5))
    s0_out, cnt_out, g1_out, g0_out = (next(it) for _ in range(4))
    acc, cacc, s1b = (next(it) for _ in range(3))
    srcb = [next(it) for _ in range(NBUF)]
    idxb = [next(it) for _ in range(NBUF)]
    dstb = [next(it) for _ in range(NBUF)]
    rows = [next(it) for _ in range(NBUF)]
    rows64, onesb = next(it), next(it)
    gsem = [next(it) for _ in range(NBUF)]
    ssem = [next(it) for _ in range(NBUF)]
    csem = [next(it) for _ in range(NBUF)]

    c = lax.axis_index("c")
    s = lax.axis_index("s")
    wid = s * NC + c
    rpt = N0 // NS

    _zero_rows(rows[0], EB, D)
    pltpu.sync_copy(rows[0], acc.at[pl.ds(s * rpt, rpt), :])
    _zero_rows(onesb, EB, CW)
    pltpu.sync_copy(onesb, cacc.at[pl.ds(s * rpt, rpt), :])
    _fill_ones(onesb, EB, CW)
    pltpu.sync_copy(s1r, s1b)
    plsc.subcore_barrier()

    per_w = E0 // NW
    base = wid * per_w
    nb = per_w // EB

    def load_b(t, b):
        off = base + t * EB
        pltpu.sync_copy(srcr.at[pl.ds(off, EB)], srcb[b])
        for j in range(EB // L):
            iv = plsc.load_gather(s1b, [srcb[b][pl.ds(j * L, L)]])
            idxb[b][pl.ds(j * L, L)] = iv
        pltpu.sync_copy(dstr.at[pl.ds(off, EB)], dstb[b])

    def wait_scatter(b):
        pltpu.make_async_copy(rows[b], acc.at[dstb[b]], ssem[b]).wait()
        pltpu.make_async_copy(onesb, cacc.at[dstb[b]], csem[b]).wait()

    def drain_issue_scatter(pb):
        pltpu.make_async_copy(feat.at[idxb[pb]], rows[pb], gsem[pb]).wait()
        pltpu.async_copy(rows[pb], acc.at[dstb[pb]], ssem[pb], add=True)
        pltpu.async_copy(onesb, cacc.at[dstb[pb]], csem[pb], add=True)

    for g in range(nb // NBUF):
        for b in range(NBUF):
            t = g * NBUF + b
            if g >= 1:
                wait_scatter(b)
            load_b(t, b)
            pltpu.async_copy(feat.at[idxb[b]], rows[b], gsem[b])
            if t >= SKEW:
                drain_issue_scatter((b - SKEW) % NBUF)
    for k in range(SKEW):
        drain_issue_scatter((nb - (SKEW - k)) % NBUF)
    for b in range(NBUF):
        wait_scatter(b)

    g1base = wid * (N1 // NW)

    def g1_issue(t, b):
        pltpu.async_copy(feat.at[s1b.at[pl.ds(g1base + t * EB, EB)]],
                         rows[b], gsem[b])

    def g1_drain(t, pb):
        pltpu.make_async_copy(feat.at[s1b.at[pl.ds(g1base + t * EB, EB)]],
                              rows[pb], gsem[pb]).wait()
        pltpu.sync_copy(rows[pb], g1_out.at[pl.ds(g1base + t * EB, EB), :])

    ng1 = (N1 // NW) // EB
    for t in range(ng1):
        g1_issue(t, t % NBUF)
        if t >= SKEW:
            g1_drain(t - SKEW, (t - SKEW) % NBUF)
    for t in range(ng1 - SKEW, ng1):
        g1_drain(t, t % NBUF)

    g0n = N0 // NW
    r0 = wid * g0n
    pltpu.sync_copy(s0r.at[pl.ds(r0, g0n)], srcb[0].at[pl.ds(0, g0n)])
    pltpu.async_copy(feat.at[srcb[0].at[pl.ds(0, g0n)]], rows64, gsem[0]).wait()
    pltpu.sync_copy(rows64, g0_out.at[pl.ds(r0, g0n), :])

    plsc.subcore_barrier()
    pltpu.sync_copy(acc.at[pl.ds(s * rpt, rpt), :],
                    s0_out.at[c, pl.ds(s * rpt, rpt), :])
    pltpu.sync_copy(cacc.at[pl.ds(s * rpt, rpt), :],
                    cnt_out.at[c, pl.ds(s * rpt, rpt), :])


_kern_b = pl.kernel(
    _kern_b_body,
    out_type=(
        jax.ShapeDtypeStruct((NC, N0, D), jnp.float32),
        jax.ShapeDtypeStruct((NC, N0, CW), jnp.float32),
        jax.ShapeDtypeStruct((N1, D), jnp.float32),
        jax.ShapeDtypeStruct((N0, D), jnp.float32),
    ),
    mesh=_MESH,
    scratch_types=(
        pltpu.VMEM_SHARED((N0, D), jnp.float32),
        pltpu.VMEM_SHARED((N0, CW), jnp.float32),
        pltpu.VMEM((N1,), jnp.int32),
    ) + tuple([pltpu.VMEM((EB,), jnp.int32)] * (3 * NBUF)) + tuple(
        [pltpu.VMEM((EB, D), jnp.float32)] * NBUF) + (
        pltpu.VMEM((N0 // NW, D), jnp.float32),
        pltpu.VMEM((EB, CW), jnp.float32),
    ) + tuple([pltpu.SemaphoreType.DMA] * (3 * NBUF)),
    compiler_params=_SC_PARAMS,
)


def _kern_c_body(*refs):
    it = iter(refs)
    h1r, srcr, dstr, s2_out, acc = (next(it) for _ in range(5))
    srcb = [next(it) for _ in range(NBUF)]
    dstb = [next(it) for _ in range(NBUF)]
    rows = [next(it) for _ in range(NBUF)]
    gsem = [next(it) for _ in range(NBUF)]
    ssem = [next(it) for _ in range(NBUF)]

    c = lax.axis_index("c")
    s = lax.axis_index("s")
    wid = s * NC + c
    rpt = N0 // NS

    _zero_rows(rows[0], EB, D)
    pltpu.sync_copy(rows[0], acc.at[pl.ds(s * rpt, rpt), :])
    plsc.subcore_barrier()

    per_w = E0 // NW
    base = wid * per_w
    nb = per_w // EB

    def wait_scatter(b):
        pltpu.make_async_copy(rows[b], acc.at[dstb[b]], ssem[b]).wait()

    def drain_issue_scatter(pb):
        pltpu.make_async_copy(h1r.at[srcb[pb]], rows[pb], gsem[pb]).wait()
        pltpu.async_copy(rows[pb], acc.at[dstb[pb]], ssem[pb], add=True)

    for g in range(nb // NBUF):
        for b in range(NBUF):
            t = g * NBUF + b
            if g >= 1:
                wait_scatter(b)
            off = base + t * EB
            pltpu.sync_copy(srcr.at[pl.ds(off, EB)], srcb[b])
            pltpu.sync_copy(dstr.at[pl.ds(off, EB)], dstb[b])
            pltpu.async_copy(h1r.at[srcb[b]], rows[b], gsem[b])
            if t >= SKEW:
                drain_issue_scatter((b - SKEW) % NBUF)
    for k in range(SKEW):
        drain_issue_scatter((nb - (SKEW - k)) % NBUF)
    for b in range(NBUF):
        wait_scatter(b)
    plsc.subcore_barrier()
    pltpu.sync_copy(acc.at[pl.ds(s * rpt, rpt), :],
                    s2_out.at[c, pl.ds(s * rpt, rpt), :])


_kern_c = pl.kernel(
    _kern_c_body,
    out_type=(jax.ShapeDtypeStruct((NC, N0, D), jnp.float32),),
    mesh=_MESH,
    scratch_types=(
        pltpu.VMEM_SHARED((N0, D), jnp.float32),
    ) + tuple([pltpu.VMEM((EB,), jnp.int32)] * (2 * NBUF)) + tuple(
        [pltpu.VMEM((EB, D), jnp.float32)] * NBUF) + tuple(
        [pltpu.SemaphoreType.DMA] * (2 * NBUF)),
    compiler_params=_SC_PARAMS,
)


def _tc_h1_body(s1r, s2r, s3r, cr, gr, wtr, br, outr):
    sfull = jnp.concatenate(
        [s1r[0] + s1r[1], s2r[0] + s2r[1], s3r[0] + s3r[1]], axis=-1)
    cnt = jnp.maximum(cr[0, :, 0:1] + cr[1, :, 0:1], 1.0)
    h = jnp.maximum(sfull / cnt + gr[...], 0.0)
    outr[...] = jnp.dot(h, wtr[...], preferred_element_type=jnp.float32) + br[...]


def _tc_h1(s1, s2, s3, cntp, g, wt, bb):
    return pl.pallas_call(
        _tc_h1_body,
        grid=(N1 // TR,),
        in_specs=[
            pl.BlockSpec((NC, TR, PW[0]), lambda i: (0, i, 0)),
            pl.BlockSpec((NC, TR, PW[1]), lambda i: (0, i, 0)),
            pl.BlockSpec((NC, TR, PW[2]), lambda i: (0, i, 0)),
            pl.BlockSpec((NC, TR, CW), lambda i: (0, i, 0)),
            pl.BlockSpec((TR, D), lambda i: (i, 0)),
            pl.BlockSpec((D, D), lambda i: (0, 0)),
            pl.BlockSpec((1, D), lambda i: (0, 0)),
        ],
        out_specs=pl.BlockSpec((TR, D), lambda i: (i, 0)),
        out_shape=jax.ShapeDtypeStruct((N1, D), jnp.float32),
    )(s1, s2, s3, cntp, g, wt, bb)


def _tc_full_body(spr, cr, gr, wtr, br, outr):
    sfull = spr[0] + spr[1]
    cnt = jnp.maximum(cr[0, :, 0:1] + cr[1, :, 0:1], 1.0)
    h = jnp.maximum(sfull / cnt + gr[...], 0.0)
    outr[...] = jnp.dot(h, wtr[...], preferred_element_type=jnp.float32) + br[...]


def _tc_full(sp, cntp, g, wt, bb, dout):
    rows_n = g.shape[0]
    return pl.pallas_call(
        _tc_full_body,
        grid=(rows_n // TR,),
        in_specs=[
            pl.BlockSpec((NC, TR, D), lambda i: (0, i, 0)),
            pl.BlockSpec((NC, TR, CW), lambda i: (0, i, 0)),
            pl.BlockSpec((TR, D), lambda i: (i, 0)),
            pl.BlockSpec((D, dout), lambda i: (0, 0)),
            pl.BlockSpec((1, dout), lambda i: (0, 0)),
        ],
        out_specs=pl.BlockSpec((TR, dout), lambda i: (i, 0)),
        out_shape=jax.ShapeDtypeStruct((rows_n, dout), jnp.float32),
    )(sp, cntp, g, wt, bb)


@jax.jit
def kernel(features, W0, b0, W1, b1, sample0, sample1, sample2,
           block0_src, block0_dst, block1_src, block1_dst):
    f1 = features[:, :PW[0]]
    f2 = features[:, PW[0]:PW[0] + PW[1]]
    f3 = features[:, PW[0] + PW[1]:]
    w0t = W0.T
    w1t = W1.T
    b0r = b0.reshape(1, -1)
    b1r = b1.reshape(1, -1)

    s0p, cnt0, g1, g0 = _kern_b(features, sample1, sample0,
                                block0_src, block0_dst)

    s1p1, idx1 = _pass_a1(f1, block1_src, block1_dst, sample2)
    (s1p2,) = _pass_a2(f2, idx1, block1_dst)
    s1p3, cnt1 = _pass_a3(f3, idx1, block1_dst)

    h1 = _tc_h1(s1p1, s1p2, s1p3, cnt1, g1, w0t, b0r)
    h0 = _tc_full(s0p, cnt0, g0, w0t, b0r, D)

    (s2p,) = _kern_c(h1, block0_src, block0_dst)
    out = _tc_full(s2p, cnt0, h0, w1t, b1r, NCLS)
    return out
